# Initial kernel scaffold; baseline (speedup 1.0000x reference)
import jax
import jax.numpy as jnp
from jax.experimental import pallas as pl


def kernel(hidden_states, Wg, W1, W3, W2):
    raise NotImplementedError("write your pallas kernel here")


# speedup vs baseline: 1.5942x; 1.5942x over previous
import functools

import jax
import jax.numpy as jnp
from jax import lax
from jax.experimental import pallas as pl
from jax.experimental.pallas import tpu as pltpu
from jax.experimental.pallas import tpu_sc as plsc

_NC = 2
_NS = 16
_NW = _NC * _NS

_TM = 128
_ROUTER_BLK = 256


def _router_body(K, x_ref, wg_ref, w_ref, i_ref):
    logits = jnp.dot(x_ref[...], wg_ref[...], preferred_element_type=jnp.float32)
    m = jnp.max(logits, axis=-1, keepdims=True)
    p = jnp.exp(logits - m)
    p = p / jnp.sum(p, axis=-1, keepdims=True)
    iota = lax.broadcasted_iota(jnp.int32, p.shape, 1)
    cur = p
    ws, idxs = [], []
    for _ in range(K):
        mx = jnp.max(cur, axis=-1, keepdims=True)
        amx = jnp.min(jnp.where(cur == mx, iota, p.shape[-1]), axis=-1,
                      keepdims=True)
        ws.append(mx)
        idxs.append(amx)
        cur = jnp.where(iota == amx, -1.0, cur)
    w_ref[...] = jnp.concatenate(ws, axis=1)
    i_ref[...] = jnp.concatenate(idxs, axis=1).astype(jnp.int32)


def _router(x, Wg, K):
    T, H = x.shape
    E = Wg.shape[1]
    blk = _ROUTER_BLK
    return pl.pallas_call(
        functools.partial(_router_body, K),
        grid=(T // blk,),
        in_specs=[
            pl.BlockSpec((blk, H), lambda i: (i, 0)),
            pl.BlockSpec((H, E), lambda i: (0, 0)),
        ],
        out_specs=[
            pl.BlockSpec((blk, K), lambda i: (i, 0)),
            pl.BlockSpec((blk, K), lambda i: (i, 0)),
        ],
        out_shape=[
            jax.ShapeDtypeStruct((T, K), jnp.float32),
            jax.ShapeDtypeStruct((T, K), jnp.int32),
        ],
    )(x, Wg)


def _routing_metadata(topk_w, topk_i, E, P, NT):
    T, K = topk_i.shape
    TK = T * K
    e = topk_i.reshape(-1)
    order = jnp.argsort(e, stable=True)
    counts = jnp.bincount(e, length=E)
    offs = jnp.concatenate([jnp.zeros((1,), jnp.int32),
                            jnp.cumsum(counts)[:-1].astype(jnp.int32)])
    pcounts = ((counts + _TM - 1) // _TM) * _TM
    poffs = jnp.concatenate([jnp.zeros((1,), jnp.int32),
                             jnp.cumsum(pcounts).astype(jnp.int32)])
    sorted_e = e[order]
    j = jnp.arange(TK, dtype=jnp.int32)
    p_sorted = j - offs[sorted_e] + poffs[sorted_e]
    tok_padded = jnp.zeros((P,), jnp.int32).at[p_sorted].set(
        (order // K).astype(jnp.int32))
    w_padded = jnp.zeros((P,), jnp.float32).at[p_sorted].set(
        topk_w.reshape(-1)[order])
    pos = jnp.zeros((TK,), jnp.int32).at[order].set(p_sorted)
    tile_start = jnp.arange(NT, dtype=jnp.int32) * _TM
    eot = jnp.clip(
        jnp.searchsorted(poffs, tile_start, side='right') - 1, 0, E - 1
    ).astype(jnp.int32)
    active = (tile_start < poffs[E]).astype(jnp.int32)
    eot = jnp.where(active == 1, eot, E - 1)
    return tok_padded, w_padded, pos, eot, active


def _dispatch_gather(x, tok_padded, P):
    H = x.shape[1]
    rows_w = P // _NW
    chunk = 48
    n_chunks = rows_w // chunk
    assert rows_w % chunk == 0
    mesh = plsc.VectorSubcoreMesh(core_axis_name="c", subcore_axis_name="s",
                                  num_cores=_NC, num_subcores=_NS)

    @functools.partial(
        pl.kernel, mesh=mesh,
        out_type=jax.ShapeDtypeStruct((P, H), jnp.float32),
        scratch_types=[
            pltpu.VMEM((rows_w,), jnp.int32),
            pltpu.VMEM((chunk, H), jnp.float32),
            pltpu.SemaphoreType.DMA,
        ],
    )
    def k(x_hbm, idx_hbm, out_hbm, idx_v, rows_v, sem):
        wid = lax.axis_index("s") * _NC + lax.axis_index("c")
        base = wid * rows_w
        pltpu.sync_copy(idx_hbm.at[pl.ds(base, rows_w)], idx_v)

        def body(c, _):
            pltpu.async_copy(
                x_hbm.at[idx_v.at[pl.ds(c * chunk, chunk)]], rows_v, sem
            ).wait()
            pltpu.sync_copy(rows_v,
                            out_hbm.at[pl.ds(base + c * chunk, chunk)])
            return _

        lax.fori_loop(0, n_chunks, body, None)

    return k(x, tok_padded)


def _ffn_body(eot_ref, act_ref, xs_ref, w_ref, w1_ref, w3_ref, w2_ref,
              out_ref):
    i = pl.program_id(0)

    @pl.when(act_ref[i] == 1)
    def _():
        x = xs_ref[...]
        g = jnp.dot(x, w1_ref[0], preferred_element_type=jnp.float32)
        u = jnp.dot(x, w3_ref[0], preferred_element_type=jnp.float32)
        h = (g * jax.nn.sigmoid(g)) * u
        y = jnp.dot(h, w2_ref[0], preferred_element_type=jnp.float32)
        out_ref[...] = y * w_ref[...]

    @pl.when(act_ref[i] == 0)
    def _():
        out_ref[...] = jnp.zeros_like(out_ref)


def _ffn(xs, w_padded, W1, W3, W2, eot, active, P, NT):
    H = xs.shape[1]
    FF = W1.shape[2]
    grid_spec = pltpu.PrefetchScalarGridSpec(
        num_scalar_prefetch=2,
        grid=(NT,),
        in_specs=[
            pl.BlockSpec((_TM, H), lambda i, eot, act: (i, 0)),
            pl.BlockSpec((_TM, 1), lambda i, eot, act: (i, 0)),
            pl.BlockSpec((1, H, FF), lambda i, eot, act: (eot[i], 0, 0)),
            pl.BlockSpec((1, H, FF), lambda i, eot, act: (eot[i], 0, 0)),
            pl.BlockSpec((1, FF, H), lambda i, eot, act: (eot[i], 0, 0)),
        ],
        out_specs=pl.BlockSpec((_TM, H), lambda i, eot, act: (i, 0)),
    )
    return pl.pallas_call(
        _ffn_body,
        grid_spec=grid_spec,
        out_shape=jax.ShapeDtypeStruct((P, H), jnp.float32),
        compiler_params=pltpu.CompilerParams(
            dimension_semantics=("arbitrary",),
        ),
    )(eot, active, xs, w_padded.reshape(P, 1), W1, W3, W2)


def _combine(ys, pos, T, K, H):
    toks_w = T // _NW
    mesh = plsc.VectorSubcoreMesh(core_axis_name="c", subcore_axis_name="s",
                                  num_cores=_NC, num_subcores=_NS)

    @functools.partial(
        pl.kernel, mesh=mesh,
        out_type=jax.ShapeDtypeStruct((T, H), jnp.float32),
        scratch_types=[
            pltpu.VMEM((toks_w * K,), jnp.int32),
            pltpu.VMEM((K, H), jnp.float32),
            pltpu.VMEM((H,), jnp.float32),
            pltpu.SemaphoreType.DMA,
        ],
    )
    def k(ys_hbm, pos_hbm, out_hbm, idx_v, rows_v, acc_v, sem):
        wid = lax.axis_index("s") * _NC + lax.axis_index("c")
        base = wid * toks_w
        pltpu.sync_copy(pos_hbm.at[pl.ds(base * K, toks_w * K)], idx_v)

        def tok_body(t, _):
            pltpu.async_copy(
                ys_hbm.at[idx_v.at[pl.ds(t * K, K)]], rows_v, sem
            ).wait()

            def col_body(i, _):
                s = rows_v[0, pl.ds(i * 16, 16)]
                for kk in range(1, K):
                    s = s + rows_v[kk, pl.ds(i * 16, 16)]
                acc_v[pl.ds(i * 16, 16)] = s
                return _

            lax.fori_loop(0, H // 16, col_body, None)
            pltpu.sync_copy(acc_v, out_hbm.at[base + t])
            return _

        lax.fori_loop(0, toks_w, tok_body, None)

    return k(ys, pos)


def kernel(hidden_states, Wg, W1, W3, W2):
    orig_shape = hidden_states.shape
    H = orig_shape[-1]
    E = Wg.shape[1]
    K = 8
    x = hidden_states.reshape(-1, H)
    T = x.shape[0]

    P = ((T * K + E * (_TM - 1) + _TM - 1) // _TM) * _TM
    NT = P // _TM

    topk_w, topk_i = _router(x, Wg, K)
    tok_padded, w_padded, pos, eot, active = _routing_metadata(
        topk_w, topk_i, E, P, NT)
    xs = _dispatch_gather(x, tok_padded, P)
    ys = _ffn(xs, w_padded, W1, W3, W2, eot, active, P, NT)
    out = _combine(ys, pos, T, K, H)
    return out.reshape(orig_shape)


# speedup vs baseline: 1.7158x; 1.0763x over previous
import functools

import jax
import jax.numpy as jnp
from jax import lax
from jax.experimental import pallas as pl
from jax.experimental.pallas import tpu as pltpu
from jax.experimental.pallas import tpu_sc as plsc

_NC = 2
_NS = 16
_NW = _NC * _NS

_TM = 128
_ROUTER_BLK = 256


def _router_body(K, x_ref, wg_ref, w_ref, i_ref):
    logits = jnp.dot(x_ref[...], wg_ref[...], preferred_element_type=jnp.float32)
    m = jnp.max(logits, axis=-1, keepdims=True)
    p = jnp.exp(logits - m)
    p = p / jnp.sum(p, axis=-1, keepdims=True)
    iota = lax.broadcasted_iota(jnp.int32, p.shape, 1)
    cur = p
    ws, idxs = [], []
    for _ in range(K):
        mx = jnp.max(cur, axis=-1, keepdims=True)
        amx = jnp.min(jnp.where(cur == mx, iota, p.shape[-1]), axis=-1,
                      keepdims=True)
        ws.append(mx)
        idxs.append(amx)
        cur = jnp.where(iota == amx, -1.0, cur)
    w_ref[...] = jnp.concatenate(ws, axis=1)
    i_ref[...] = jnp.concatenate(idxs, axis=1).astype(jnp.int32)


def _router(x, Wg, K):
    T, H = x.shape
    E = Wg.shape[1]
    blk = _ROUTER_BLK
    return pl.pallas_call(
        functools.partial(_router_body, K),
        grid=(T // blk,),
        in_specs=[
            pl.BlockSpec((blk, H), lambda i: (i, 0)),
            pl.BlockSpec((H, E), lambda i: (0, 0)),
        ],
        out_specs=[
            pl.BlockSpec((blk, K), lambda i: (i, 0)),
            pl.BlockSpec((blk, K), lambda i: (i, 0)),
        ],
        out_shape=[
            jax.ShapeDtypeStruct((T, K), jnp.float32),
            jax.ShapeDtypeStruct((T, K), jnp.int32),
        ],
    )(x, Wg)


def _routing_metadata(topk_w, topk_i, E, P, NT):
    T, K = topk_i.shape
    TK = T * K
    e = topk_i.reshape(-1)
    pair = jnp.arange(TK, dtype=jnp.int32)
    w_flat = topk_w.reshape(-1)
    sorted_e, order, w_sorted = jax.lax.sort((e, pair, w_flat), num_keys=1)
    offs = jnp.searchsorted(
        sorted_e, jnp.arange(E, dtype=jnp.int32), side='left'
    ).astype(jnp.int32)
    counts = jnp.diff(jnp.concatenate(
        [offs, jnp.full((1,), TK, jnp.int32)]))
    pcounts = ((counts + _TM - 1) // _TM) * _TM
    poffs = jnp.concatenate([jnp.zeros((1,), jnp.int32),
                             jnp.cumsum(pcounts).astype(jnp.int32)])
    padshift = poffs[:E] - offs
    p_sorted = pair + padshift[sorted_e]
    tok_padded = jnp.zeros((P,), jnp.int32).at[p_sorted].set(
        (order // K).astype(jnp.int32), mode="promise_in_bounds",
        unique_indices=True)
    w_padded = jnp.zeros((P,), jnp.float32).at[p_sorted].set(
        w_sorted, mode="promise_in_bounds", unique_indices=True)
    _, pos = jax.lax.sort((order, p_sorted), num_keys=1)
    tile_start = jnp.arange(NT, dtype=jnp.int32) * _TM
    eot = jnp.clip(
        jnp.searchsorted(poffs, tile_start, side='right') - 1, 0, E - 1
    ).astype(jnp.int32)
    active = (tile_start < poffs[E]).astype(jnp.int32)
    eot = jnp.where(active == 1, eot, E - 1)
    return tok_padded, w_padded, pos, eot, active


def _dispatch_gather(x, tok_padded, P):
    H = x.shape[1]
    rows_w = P // _NW
    chunk = 24
    n_chunks = rows_w // chunk
    assert rows_w % chunk == 0
    mesh = plsc.VectorSubcoreMesh(core_axis_name="c", subcore_axis_name="s",
                                  num_cores=_NC, num_subcores=_NS)

    @functools.partial(
        pl.kernel, mesh=mesh,
        out_type=jax.ShapeDtypeStruct((P, H), jnp.float32),
        scratch_types=[
            pltpu.VMEM((rows_w,), jnp.int32),
            pltpu.VMEM((chunk, H), jnp.float32),
            pltpu.VMEM((chunk, H), jnp.float32),
            pltpu.SemaphoreType.DMA,
            pltpu.SemaphoreType.DMA,
            pltpu.SemaphoreType.DMA,
            pltpu.SemaphoreType.DMA,
        ],
    )
    def k(x_hbm, idx_hbm, out_hbm, idx_v, buf0, buf1, gs0, gs1, ss0, ss1):
        wid = lax.axis_index("s") * _NC + lax.axis_index("c")
        base = wid * rows_w
        pltpu.sync_copy(idx_hbm.at[pl.ds(base, rows_w)], idx_v)
        bufs = (buf0, buf1)
        gsems = (gs0, gs1)
        ssems = (ss0, ss1)
        gd = [None] * n_chunks
        sd = [None] * n_chunks
        gd[0] = pltpu.async_copy(
            x_hbm.at[idx_v.at[pl.ds(0, chunk)]], bufs[0], gsems[0])
        for c in range(n_chunks):
            b = c & 1
            gd[c].wait()
            if c >= 1:
                sd[c - 1].wait()
            if c + 1 < n_chunks:
                gd[c + 1] = pltpu.async_copy(
                    x_hbm.at[idx_v.at[pl.ds((c + 1) * chunk, chunk)]],
                    bufs[1 - b], gsems[1 - b])
            sd[c] = pltpu.async_copy(
                bufs[b], out_hbm.at[pl.ds(base + c * chunk, chunk)], ssems[b])
        sd[n_chunks - 1].wait()

    return k(x, tok_padded)


def _ffn_body(eot_ref, act_ref, xs_ref, w_ref, w1_ref, w3_ref, w2_ref,
              out_ref):
    i = pl.program_id(0)

    @pl.when(act_ref[i] == 1)
    def _():
        x = xs_ref[...]
        g = jnp.dot(x, w1_ref[0], preferred_element_type=jnp.float32)
        u = jnp.dot(x, w3_ref[0], preferred_element_type=jnp.float32)
        h = (g * jax.nn.sigmoid(g)) * u
        y = jnp.dot(h, w2_ref[0], preferred_element_type=jnp.float32)
        out_ref[...] = y * w_ref[...]

    @pl.when(act_ref[i] == 0)
    def _():
        out_ref[...] = jnp.zeros_like(out_ref)


def _ffn(xs, w_padded, W1, W3, W2, eot, active, P, NT):
    H = xs.shape[1]
    FF = W1.shape[2]
    grid_spec = pltpu.PrefetchScalarGridSpec(
        num_scalar_prefetch=2,
        grid=(NT,),
        in_specs=[
            pl.BlockSpec((_TM, H), lambda i, eot, act: (i, 0)),
            pl.BlockSpec((_TM, 1), lambda i, eot, act: (i, 0)),
            pl.BlockSpec((1, H, FF), lambda i, eot, act: (eot[i], 0, 0)),
            pl.BlockSpec((1, H, FF), lambda i, eot, act: (eot[i], 0, 0)),
            pl.BlockSpec((1, FF, H), lambda i, eot, act: (eot[i], 0, 0)),
        ],
        out_specs=pl.BlockSpec((_TM, H), lambda i, eot, act: (i, 0)),
    )
    return pl.pallas_call(
        _ffn_body,
        grid_spec=grid_spec,
        out_shape=jax.ShapeDtypeStruct((P, H), jnp.float32),
        compiler_params=pltpu.CompilerParams(
            dimension_semantics=("arbitrary",),
        ),
    )(eot, active, xs, w_padded.reshape(P, 1), W1, W3, W2)


def _combine(ys, pos, T, K, H):
    toks_w = T // _NW
    mesh = plsc.VectorSubcoreMesh(core_axis_name="c", subcore_axis_name="s",
                                  num_cores=_NC, num_subcores=_NS)

    @functools.partial(
        pl.kernel, mesh=mesh,
        out_type=jax.ShapeDtypeStruct((T, H), jnp.float32),
        scratch_types=[
            pltpu.VMEM((toks_w * K,), jnp.int32),
            pltpu.VMEM((K, H), jnp.float32),
            pltpu.VMEM((H,), jnp.float32),
            pltpu.SemaphoreType.DMA,
        ],
    )
    def k(ys_hbm, pos_hbm, out_hbm, idx_v, rows_v, acc_v, sem):
        wid = lax.axis_index("s") * _NC + lax.axis_index("c")
        base = wid * toks_w
        pltpu.sync_copy(pos_hbm.at[pl.ds(base * K, toks_w * K)], idx_v)

        def tok_body(t, _):
            pltpu.async_copy(
                ys_hbm.at[idx_v.at[pl.ds(t * K, K)]], rows_v, sem
            ).wait()

            def col_body(i, _):
                s = rows_v[0, pl.ds(i * 16, 16)]
                for kk in range(1, K):
                    s = s + rows_v[kk, pl.ds(i * 16, 16)]
                acc_v[pl.ds(i * 16, 16)] = s
                return _

            lax.fori_loop(0, H // 16, col_body, None)
            pltpu.sync_copy(acc_v, out_hbm.at[base + t])
            return _

        lax.fori_loop(0, toks_w, tok_body, None)

    return k(ys, pos)


def kernel(hidden_states, Wg, W1, W3, W2):
    orig_shape = hidden_states.shape
    H = orig_shape[-1]
    E = Wg.shape[1]
    K = 8
    x = hidden_states.reshape(-1, H)
    T = x.shape[0]

    P = ((T * K + E * (_TM - 1) + _TM - 1) // _TM) * _TM
    NT = P // _TM

    topk_w, topk_i = _router(x, Wg, K)
    tok_padded, w_padded, pos, eot, active = _routing_metadata(
        topk_w, topk_i, E, P, NT)
    xs = _dispatch_gather(x, tok_padded, P)
    ys = _ffn(xs, w_padded, W1, W3, W2, eot, active, P, NT)
    out = _combine(ys, pos, T, K, H)
    return out.reshape(orig_shape)


# speedup vs baseline: 2.5486x; 1.4853x over previous
import functools

import jax
import jax.numpy as jnp
from jax import lax
from jax.experimental import pallas as pl
from jax.experimental.pallas import tpu as pltpu
from jax.experimental.pallas import tpu_sc as plsc

_NC = 2
_NS = 16
_NW = _NC * _NS

_TM = 128
_ROUTER_BLK = 256


def _router_body(K, x_ref, wg_ref, w_ref, i_ref):
    logits = jnp.dot(x_ref[...], wg_ref[...], preferred_element_type=jnp.float32)
    m = jnp.max(logits, axis=-1, keepdims=True)
    p = jnp.exp(logits - m)
    p = p / jnp.sum(p, axis=-1, keepdims=True)
    iota = lax.broadcasted_iota(jnp.int32, p.shape, 1)
    cur = p
    ws, idxs = [], []
    for _ in range(K):
        mx = jnp.max(cur, axis=-1, keepdims=True)
        amx = jnp.min(jnp.where(cur == mx, iota, p.shape[-1]), axis=-1,
                      keepdims=True)
        ws.append(mx)
        idxs.append(amx)
        cur = jnp.where(iota == amx, -1.0, cur)
    w_ref[...] = jnp.concatenate(ws, axis=1)
    i_ref[...] = jnp.concatenate(idxs, axis=1).astype(jnp.int32)


def _router(x, Wg, K):
    T, H = x.shape
    E = Wg.shape[1]
    blk = _ROUTER_BLK
    return pl.pallas_call(
        functools.partial(_router_body, K),
        grid=(T // blk,),
        in_specs=[
            pl.BlockSpec((blk, H), lambda i: (i, 0)),
            pl.BlockSpec((H, E), lambda i: (0, 0)),
        ],
        out_specs=[
            pl.BlockSpec((blk, K), lambda i: (i, 0)),
            pl.BlockSpec((blk, K), lambda i: (i, 0)),
        ],
        out_shape=[
            jax.ShapeDtypeStruct((T, K), jnp.float32),
            jax.ShapeDtypeStruct((T, K), jnp.int32),
        ],
    )(x, Wg)


def _routing_metadata(topk_w, topk_i, E, P, NT):
    T, K = topk_i.shape
    TK = T * K
    e = topk_i.reshape(-1)
    pair = jnp.arange(TK, dtype=jnp.int32)
    w_flat = topk_w.reshape(-1)
    sorted_e, order, w_sorted = jax.lax.sort((e, pair, w_flat), num_keys=1)
    offs = jnp.searchsorted(
        sorted_e, jnp.arange(E, dtype=jnp.int32), side='left'
    ).astype(jnp.int32)
    counts = jnp.diff(jnp.concatenate(
        [offs, jnp.full((1,), TK, jnp.int32)]))
    pcounts = ((counts + _TM - 1) // _TM) * _TM
    poffs = jnp.concatenate([jnp.zeros((1,), jnp.int32),
                             jnp.cumsum(pcounts).astype(jnp.int32)])
    padshift = poffs[:E] - offs
    p_sorted = pair + padshift[sorted_e]
    pad_idx = jnp.arange(P, dtype=jnp.int32) % T
    tok_padded = pad_idx.at[p_sorted].set(
        (order // K).astype(jnp.int32), mode="promise_in_bounds",
        unique_indices=True)
    w_padded = jnp.zeros((P,), jnp.float32).at[p_sorted].set(
        w_sorted, mode="promise_in_bounds", unique_indices=True)
    _, pos = jax.lax.sort((order, p_sorted), num_keys=1)
    tile_start = jnp.arange(NT, dtype=jnp.int32) * _TM
    eot = jnp.clip(
        jnp.searchsorted(poffs, tile_start, side='right') - 1, 0, E - 1
    ).astype(jnp.int32)
    active = (tile_start < poffs[E]).astype(jnp.int32)
    eot = jnp.where(active == 1, eot, E - 1)
    return tok_padded, w_padded, pos, eot, active


def _dispatch_gather(x, tok_padded, P):
    H = x.shape[1]
    rows_w = P // _NW
    chunk = 24
    n_chunks = rows_w // chunk
    assert rows_w % chunk == 0
    mesh = plsc.VectorSubcoreMesh(core_axis_name="c", subcore_axis_name="s",
                                  num_cores=_NC, num_subcores=_NS)

    @functools.partial(
        pl.kernel, mesh=mesh,
        out_type=jax.ShapeDtypeStruct((P, H), jnp.float32),
        scratch_types=[
            pltpu.VMEM((rows_w,), jnp.int32),
            pltpu.VMEM((chunk, H), jnp.float32),
            pltpu.VMEM((chunk, H), jnp.float32),
            pltpu.SemaphoreType.DMA,
            pltpu.SemaphoreType.DMA,
            pltpu.SemaphoreType.DMA,
            pltpu.SemaphoreType.DMA,
        ],
    )
    def k(x_hbm, idx_hbm, out_hbm, idx_v, buf0, buf1, gs0, gs1, ss0, ss1):
        wid = lax.axis_index("s") * _NC + lax.axis_index("c")
        base = wid * rows_w
        pltpu.sync_copy(idx_hbm.at[pl.ds(base, rows_w)], idx_v)
        bufs = (buf0, buf1)
        gsems = (gs0, gs1)
        ssems = (ss0, ss1)
        gd = [None] * n_chunks
        sd = [None] * n_chunks
        gd[0] = pltpu.async_copy(
            x_hbm.at[idx_v.at[pl.ds(0, chunk)]], bufs[0], gsems[0])
        for c in range(n_chunks):
            b = c & 1
            gd[c].wait()
            if c >= 1:
                sd[c - 1].wait()
            if c + 1 < n_chunks:
                gd[c + 1] = pltpu.async_copy(
                    x_hbm.at[idx_v.at[pl.ds((c + 1) * chunk, chunk)]],
                    bufs[1 - b], gsems[1 - b])
            sd[c] = pltpu.async_copy(
                bufs[b], out_hbm.at[pl.ds(base + c * chunk, chunk)], ssems[b])
        sd[n_chunks - 1].wait()

    return k(x, tok_padded)


def _ffn_body(eot_ref, act_ref, xs_ref, w_ref, w1_ref, w3_ref, w2_ref,
              out_ref):
    i = pl.program_id(0)

    @pl.when(act_ref[i] == 1)
    def _():
        x = xs_ref[...]
        g = jnp.dot(x, w1_ref[0], preferred_element_type=jnp.float32)
        u = jnp.dot(x, w3_ref[0], preferred_element_type=jnp.float32)
        h = (g * jax.nn.sigmoid(g)) * u
        y = jnp.dot(h, w2_ref[0], preferred_element_type=jnp.float32)
        out_ref[...] = y * w_ref[...]

    @pl.when(act_ref[i] == 0)
    def _():
        out_ref[...] = jnp.zeros_like(out_ref)


def _ffn(xs, w_padded, W1, W3, W2, eot, active, P, NT):
    H = xs.shape[1]
    FF = W1.shape[2]
    grid_spec = pltpu.PrefetchScalarGridSpec(
        num_scalar_prefetch=2,
        grid=(NT,),
        in_specs=[
            pl.BlockSpec((_TM, H), lambda i, eot, act: (i, 0)),
            pl.BlockSpec((_TM, 1), lambda i, eot, act: (i, 0)),
            pl.BlockSpec((1, H, FF), lambda i, eot, act: (eot[i], 0, 0)),
            pl.BlockSpec((1, H, FF), lambda i, eot, act: (eot[i], 0, 0)),
            pl.BlockSpec((1, FF, H), lambda i, eot, act: (eot[i], 0, 0)),
        ],
        out_specs=pl.BlockSpec((_TM, H), lambda i, eot, act: (i, 0)),
    )
    return pl.pallas_call(
        _ffn_body,
        grid_spec=grid_spec,
        out_shape=jax.ShapeDtypeStruct((P, H), jnp.float32),
        compiler_params=pltpu.CompilerParams(
            dimension_semantics=("arbitrary",),
        ),
    )(eot, active, xs, w_padded.reshape(P, 1), W1, W3, W2)


def _combine(ys, pos, T, K, H):
    toks_w = T // _NW
    mesh = plsc.VectorSubcoreMesh(core_axis_name="c", subcore_axis_name="s",
                                  num_cores=_NC, num_subcores=_NS)

    @functools.partial(
        pl.kernel, mesh=mesh,
        out_type=jax.ShapeDtypeStruct((T, H), jnp.float32),
        scratch_types=[
            pltpu.VMEM((toks_w * K,), jnp.int32),
            pltpu.VMEM((K, H), jnp.float32),
            pltpu.VMEM((H,), jnp.float32),
            pltpu.SemaphoreType.DMA,
        ],
    )
    def k(ys_hbm, pos_hbm, out_hbm, idx_v, rows_v, acc_v, sem):
        wid = lax.axis_index("s") * _NC + lax.axis_index("c")
        base = wid * toks_w
        pltpu.sync_copy(pos_hbm.at[pl.ds(base * K, toks_w * K)], idx_v)

        def tok_body(t, _):
            pltpu.async_copy(
                ys_hbm.at[idx_v.at[pl.ds(t * K, K)]], rows_v, sem
            ).wait()

            def col_body(i, _):
                s = rows_v[0, pl.ds(i * 16, 16)]
                for kk in range(1, K):
                    s = s + rows_v[kk, pl.ds(i * 16, 16)]
                acc_v[pl.ds(i * 16, 16)] = s
                return _

            lax.fori_loop(0, H // 16, col_body, None)
            pltpu.sync_copy(acc_v, out_hbm.at[base + t])
            return _

        lax.fori_loop(0, toks_w, tok_body, None)

    return k(ys, pos)


def kernel(hidden_states, Wg, W1, W3, W2):
    orig_shape = hidden_states.shape
    H = orig_shape[-1]
    E = Wg.shape[1]
    K = 8
    x = hidden_states.reshape(-1, H)
    T = x.shape[0]

    P = ((T * K + E * (_TM - 1) + _TM - 1) // _TM) * _TM
    NT = P // _TM

    topk_w, topk_i = _router(x, Wg, K)
    tok_padded, w_padded, pos, eot, active = _routing_metadata(
        topk_w, topk_i, E, P, NT)
    xs = _dispatch_gather(x, tok_padded, P)
    ys = _ffn(xs, w_padded, W1, W3, W2, eot, active, P, NT)
    out = _combine(ys, pos, T, K, H)
    return out.reshape(orig_shape)


# speedup vs baseline: 2.7668x; 1.0856x over previous
import functools

import jax
import jax.numpy as jnp
from jax import lax
from jax.experimental import pallas as pl
from jax.experimental.pallas import tpu as pltpu
from jax.experimental.pallas import tpu_sc as plsc

_NC = 2
_NS = 16
_NW = _NC * _NS

_TM = 128
_ROUTER_BLK = 256


def _router_body(K, x_ref, wg_ref, w_ref, i_ref):
    logits = jnp.dot(x_ref[...], wg_ref[...], preferred_element_type=jnp.float32)
    m = jnp.max(logits, axis=-1, keepdims=True)
    p = jnp.exp(logits - m)
    p = p / jnp.sum(p, axis=-1, keepdims=True)
    iota = lax.broadcasted_iota(jnp.int32, p.shape, 1)
    cur = p
    ws, idxs = [], []
    for _ in range(K):
        mx = jnp.max(cur, axis=-1, keepdims=True)
        amx = jnp.min(jnp.where(cur == mx, iota, p.shape[-1]), axis=-1,
                      keepdims=True)
        ws.append(mx)
        idxs.append(amx)
        cur = jnp.where(iota == amx, -1.0, cur)
    w_ref[...] = jnp.concatenate(ws, axis=1)
    i_ref[...] = jnp.concatenate(idxs, axis=1).astype(jnp.int32)


def _router(x, Wg, K):
    T, H = x.shape
    E = Wg.shape[1]
    blk = _ROUTER_BLK
    return pl.pallas_call(
        functools.partial(_router_body, K),
        grid=(T // blk,),
        in_specs=[
            pl.BlockSpec((blk, H), lambda i: (i, 0)),
            pl.BlockSpec((H, E), lambda i: (0, 0)),
        ],
        out_specs=[
            pl.BlockSpec((blk, K), lambda i: (i, 0)),
            pl.BlockSpec((blk, K), lambda i: (i, 0)),
        ],
        out_shape=[
            jax.ShapeDtypeStruct((T, K), jnp.float32),
            jax.ShapeDtypeStruct((T, K), jnp.int32),
        ],
    )(x, Wg)


def _routing_metadata(topk_w, topk_i, E, P, NT):
    T, K = topk_i.shape
    TK = T * K
    e = topk_i.reshape(-1)
    pair = jnp.arange(TK, dtype=jnp.int32)
    w_flat = topk_w.reshape(-1)
    sorted_e, order, w_sorted = jax.lax.sort((e, pair, w_flat), num_keys=1)
    offs = jnp.searchsorted(
        sorted_e, jnp.arange(E, dtype=jnp.int32), side='left'
    ).astype(jnp.int32)
    counts = jnp.diff(jnp.concatenate(
        [offs, jnp.full((1,), TK, jnp.int32)]))
    pcounts = ((counts + _TM - 1) // _TM) * _TM
    poffs = jnp.concatenate([jnp.zeros((1,), jnp.int32),
                             jnp.cumsum(pcounts).astype(jnp.int32)])
    padshift = poffs[:E] - offs
    p_sorted = pair + padshift[sorted_e]
    pad_idx = jnp.arange(P, dtype=jnp.int32) % T
    tok_padded = pad_idx.at[p_sorted].set(
        (order // K).astype(jnp.int32), mode="promise_in_bounds",
        unique_indices=True)
    w_padded = jnp.zeros((P,), jnp.float32).at[p_sorted].set(
        w_sorted, mode="promise_in_bounds", unique_indices=True)
    _, pos = jax.lax.sort((order, p_sorted), num_keys=1)
    tile_start = jnp.arange(NT, dtype=jnp.int32) * _TM
    eot = jnp.clip(
        jnp.searchsorted(poffs, tile_start, side='right') - 1, 0, E - 1
    ).astype(jnp.int32)
    active = (tile_start < poffs[E]).astype(jnp.int32)
    eot = jnp.where(active == 1, eot, E - 1)
    n_active = jnp.maximum(jnp.sum(active), 1)
    xsblk = jnp.where(active == 1, jnp.arange(NT, dtype=jnp.int32),
                      n_active - 1).astype(jnp.int32)
    return tok_padded, w_padded, pos, eot, active, xsblk


def _dispatch_gather(x, tok_padded, P):
    H = x.shape[1]
    rows_w = P // _NW
    chunk = 24
    n_chunks = rows_w // chunk
    assert rows_w % chunk == 0
    mesh = plsc.VectorSubcoreMesh(core_axis_name="c", subcore_axis_name="s",
                                  num_cores=_NC, num_subcores=_NS)

    @functools.partial(
        pl.kernel, mesh=mesh,
        out_type=jax.ShapeDtypeStruct((P, H), jnp.float32),
        scratch_types=[
            pltpu.VMEM((rows_w,), jnp.int32),
            pltpu.VMEM((chunk, H), jnp.float32),
            pltpu.VMEM((chunk, H), jnp.float32),
            pltpu.SemaphoreType.DMA,
            pltpu.SemaphoreType.DMA,
            pltpu.SemaphoreType.DMA,
            pltpu.SemaphoreType.DMA,
        ],
    )
    def k(x_hbm, idx_hbm, out_hbm, idx_v, buf0, buf1, gs0, gs1, ss0, ss1):
        wid = lax.axis_index("s") * _NC + lax.axis_index("c")
        base = wid * rows_w
        pltpu.sync_copy(idx_hbm.at[pl.ds(base, rows_w)], idx_v)
        bufs = (buf0, buf1)
        gsems = (gs0, gs1)
        ssems = (ss0, ss1)
        gd = [None] * n_chunks
        sd = [None] * n_chunks
        gd[0] = pltpu.async_copy(
            x_hbm.at[idx_v.at[pl.ds(0, chunk)]], bufs[0], gsems[0])
        for c in range(n_chunks):
            b = c & 1
            gd[c].wait()
            if c >= 1:
                sd[c - 1].wait()
            if c + 1 < n_chunks:
                gd[c + 1] = pltpu.async_copy(
                    x_hbm.at[idx_v.at[pl.ds((c + 1) * chunk, chunk)]],
                    bufs[1 - b], gsems[1 - b])
            sd[c] = pltpu.async_copy(
                bufs[b], out_hbm.at[pl.ds(base + c * chunk, chunk)], ssems[b])
        sd[n_chunks - 1].wait()

    return k(x, tok_padded)


def _ffn_body(eot_ref, act_ref, xsblk_ref, xs_ref, w_ref, w1_ref, w3_ref,
              w2_ref, out_ref):
    i = pl.program_id(0)

    @pl.when(act_ref[i] == 1)
    def _():
        x = xs_ref[...].astype(jnp.float32)
        g = jnp.dot(x, w1_ref[0], preferred_element_type=jnp.float32)
        u = jnp.dot(x, w3_ref[0], preferred_element_type=jnp.float32)
        h = (g * jax.nn.sigmoid(g)) * u
        y = jnp.dot(h, w2_ref[0], preferred_element_type=jnp.float32)
        out_ref[...] = y * w_ref[...]

    @pl.when(act_ref[i] == 0)
    def _():
        out_ref[...] = jnp.zeros_like(out_ref)


def _ffn(xs, w_padded, W1, W3, W2, eot, active, xsblk, P, NT):
    H = xs.shape[1]
    FF = W1.shape[2]
    grid_spec = pltpu.PrefetchScalarGridSpec(
        num_scalar_prefetch=3,
        grid=(NT,),
        in_specs=[
            pl.BlockSpec((_TM, H), lambda i, eot, act, xb: (xb[i], 0)),
            pl.BlockSpec((_TM, 1), lambda i, eot, act, xb: (xb[i], 0)),
            pl.BlockSpec((1, H, FF), lambda i, eot, act, xb: (eot[i], 0, 0)),
            pl.BlockSpec((1, H, FF), lambda i, eot, act, xb: (eot[i], 0, 0)),
            pl.BlockSpec((1, FF, H), lambda i, eot, act, xb: (eot[i], 0, 0)),
        ],
        out_specs=pl.BlockSpec((_TM, H), lambda i, eot, act, xb: (i, 0)),
    )
    return pl.pallas_call(
        _ffn_body,
        grid_spec=grid_spec,
        out_shape=jax.ShapeDtypeStruct((P, H), jnp.float32),
        compiler_params=pltpu.CompilerParams(
            dimension_semantics=("arbitrary",),
        ),
    )(eot, active, xsblk, xs, w_padded.reshape(P, 1), W1, W3, W2)


def _combine(ys, pos, T, K, H):
    toks_w = T // _NW
    BT = 2
    rounds = toks_w // BT
    mesh = plsc.VectorSubcoreMesh(core_axis_name="c", subcore_axis_name="s",
                                  num_cores=_NC, num_subcores=_NS)

    @functools.partial(
        pl.kernel, mesh=mesh,
        out_type=jax.ShapeDtypeStruct((T, H), jnp.float32),
        scratch_types=[
            pltpu.VMEM((toks_w * K,), jnp.int32),
            pltpu.VMEM((BT * K, H), jnp.float32),
            pltpu.VMEM((BT * K, H), jnp.float32),
            pltpu.VMEM((BT, H), jnp.float32),
            pltpu.VMEM((BT, H), jnp.float32),
            pltpu.SemaphoreType.DMA,
            pltpu.SemaphoreType.DMA,
            pltpu.SemaphoreType.DMA,
            pltpu.SemaphoreType.DMA,
        ],
    )
    def k(ys_hbm, pos_hbm, out_hbm, idx_v, b0, b1, o0, o1,
          gs0, gs1, ss0, ss1):
        wid = lax.axis_index("s") * _NC + lax.axis_index("c")
        base = wid * toks_w
        pltpu.sync_copy(pos_hbm.at[pl.ds(base * K, toks_w * K)], idx_v)

        def gsrc(r):
            return ys_hbm.at[idx_v.at[pl.ds(r * BT * K, BT * K)]]

        def odst(r):
            return out_hbm.at[pl.ds(base + r * BT, BT)]

        def accum(buf, ost):
            def col_body(i, _):
                for tt in range(BT):
                    s = buf[tt * K, pl.ds(i * 16, 16)]
                    for kk in range(1, K):
                        s = s + buf[tt * K + kk, pl.ds(i * 16, 16)]
                    ost[tt, pl.ds(i * 16, 16)] = s
                return _
            lax.fori_loop(0, H // 16, col_body, None)

        pltpu.async_copy(gsrc(0), b0, gs0)
        pltpu.async_copy(gsrc(1), b1, gs1)

        def body(i, _):
            for par, (buf, ost, gs, ss) in enumerate(
                    ((b0, o0, gs0, ss0), (b1, o1, gs1, ss1))):
                r = 2 * i + par
                pltpu.make_async_copy(gsrc(r), buf, gs).wait()

                @pl.when(i > 0)
                def _():
                    pltpu.make_async_copy(ost, odst(r), ss).wait()

                accum(buf, ost)
                pltpu.async_copy(ost, odst(r), ss)

                @pl.when(r + 2 < rounds)
                def _():
                    pltpu.async_copy(gsrc(r + 2), buf, gs)
            return _

        lax.fori_loop(0, rounds // 2, body, None)
        pltpu.make_async_copy(o0, odst(rounds - 2), ss0).wait()
        pltpu.make_async_copy(o1, odst(rounds - 1), ss1).wait()

    return k(ys, pos)


def kernel(hidden_states, Wg, W1, W3, W2):
    orig_shape = hidden_states.shape
    H = orig_shape[-1]
    E = Wg.shape[1]
    K = 8
    x = hidden_states.reshape(-1, H)
    T = x.shape[0]

    P = ((T * K + E * (_TM - 1) + _TM - 1) // _TM) * _TM
    NT = P // _TM

    topk_w, topk_i = _router(x, Wg, K)
    tok_padded, w_padded, pos, eot, active, xsblk = _routing_metadata(
        topk_w, topk_i, E, P, NT)
    xs = _dispatch_gather(x, tok_padded, P)
    ys = _ffn(xs, w_padded, W1, W3, W2, eot, active, xsblk, P, NT)
    out = _combine(ys, pos, T, K, H)
    return out.reshape(orig_shape)


# speedup vs baseline: 2.9983x; 1.0837x over previous
import functools

import jax
import jax.numpy as jnp
from jax import lax
from jax.experimental import pallas as pl
from jax.experimental.pallas import tpu as pltpu
from jax.experimental.pallas import tpu_sc as plsc

_NC = 2
_NS = 16
_NW = _NC * _NS

_TM = 128
_ROUTER_BLK = 256


def _router_body(K, x_ref, wg_ref, w_ref, i_ref):
    logits = jnp.dot(x_ref[...], wg_ref[...], preferred_element_type=jnp.float32)
    m = jnp.max(logits, axis=-1, keepdims=True)
    p = jnp.exp(logits - m)
    p = p / jnp.sum(p, axis=-1, keepdims=True)
    iota = lax.broadcasted_iota(jnp.int32, p.shape, 1)
    cur = p
    ws, idxs = [], []
    for _ in range(K):
        mx = jnp.max(cur, axis=-1, keepdims=True)
        amx = jnp.min(jnp.where(cur == mx, iota, p.shape[-1]), axis=-1,
                      keepdims=True)
        ws.append(mx)
        idxs.append(amx)
        cur = jnp.where(iota == amx, -1.0, cur)
    w_ref[...] = jnp.concatenate(ws, axis=1)
    i_ref[...] = jnp.concatenate(idxs, axis=1).astype(jnp.int32)


def _router(x, Wg, K):
    T, H = x.shape
    E = Wg.shape[1]
    blk = _ROUTER_BLK
    return pl.pallas_call(
        functools.partial(_router_body, K),
        grid=(T // blk,),
        in_specs=[
            pl.BlockSpec((blk, H), lambda i: (i, 0)),
            pl.BlockSpec((H, E), lambda i: (0, 0)),
        ],
        out_specs=[
            pl.BlockSpec((blk, K), lambda i: (i, 0)),
            pl.BlockSpec((blk, K), lambda i: (i, 0)),
        ],
        out_shape=[
            jax.ShapeDtypeStruct((T, K), jnp.float32),
            jax.ShapeDtypeStruct((T, K), jnp.int32),
        ],
    )(x, Wg)


def _routing_metadata(topk_w, topk_i, E, P, NT):
    T, K = topk_i.shape
    TK = T * K
    e = topk_i.reshape(-1)
    pair = jnp.arange(TK, dtype=jnp.int32)
    w_flat = topk_w.reshape(-1)
    sorted_e, order, w_sorted = jax.lax.sort((e, pair, w_flat), num_keys=1)
    offs = jnp.searchsorted(
        sorted_e, jnp.arange(E, dtype=jnp.int32), side='left'
    ).astype(jnp.int32)
    counts = jnp.diff(jnp.concatenate(
        [offs, jnp.full((1,), TK, jnp.int32)]))
    pcounts = ((counts + _TM - 1) // _TM) * _TM
    poffs = jnp.concatenate([jnp.zeros((1,), jnp.int32),
                             jnp.cumsum(pcounts).astype(jnp.int32)])
    padshift = poffs[:E] - offs
    p_sorted = pair + padshift[sorted_e]
    tok_sorted = (order // K).astype(jnp.int32)
    w_padded = jnp.zeros((P,), jnp.float32).at[p_sorted].set(
        w_sorted, mode="promise_in_bounds", unique_indices=True)
    _, pos = jax.lax.sort((order, p_sorted), num_keys=1)
    tile_start = jnp.arange(NT, dtype=jnp.int32) * _TM
    eot = jnp.clip(
        jnp.searchsorted(poffs, tile_start, side='right') - 1, 0, E - 1
    ).astype(jnp.int32)
    active = (tile_start < poffs[E]).astype(jnp.int32)
    eot = jnp.where(active == 1, eot, E - 1)
    n_active = jnp.maximum(jnp.sum(active), 1)
    xsblk = jnp.where(active == 1, jnp.arange(NT, dtype=jnp.int32),
                      n_active - 1).astype(jnp.int32)
    return tok_sorted, sorted_e, padshift, w_padded, pos, eot, active, xsblk


def _dispatch_gather(x, tok_sorted, sorted_e, padshift, P, E):
    H = x.shape[1]
    TK = tok_sorted.shape[0]
    pairs_w = TK // _NW
    chunk = 16
    n_chunks = pairs_w // chunk
    mesh = plsc.VectorSubcoreMesh(core_axis_name="c", subcore_axis_name="s",
                                  num_cores=_NC, num_subcores=_NS)

    @functools.partial(
        pl.kernel, mesh=mesh,
        out_type=jax.ShapeDtypeStruct((P, H), jnp.float32),
        compiler_params=pltpu.CompilerParams(needs_layout_passes=False),
        scratch_types=[
            pltpu.VMEM((pairs_w,), jnp.int32),
            pltpu.VMEM((pairs_w,), jnp.int32),
            pltpu.VMEM((E,), jnp.int32),
            pltpu.VMEM((chunk, H), jnp.float32),
            pltpu.VMEM((chunk, H), jnp.float32),
            pltpu.SemaphoreType.DMA,
            pltpu.SemaphoreType.DMA,
            pltpu.SemaphoreType.DMA,
            pltpu.SemaphoreType.DMA,
        ],
    )
    def k(x_hbm, tok_hbm, e_hbm, ps_hbm, out_hbm, tok_v, e_v, ps_v,
          buf0, buf1, gs0, gs1, ss0, ss1):
        wid = lax.axis_index("s") * _NC + lax.axis_index("c")
        j0 = wid * pairs_w
        pltpu.sync_copy(tok_hbm.at[pl.ds(j0, pairs_w)], tok_v)
        pltpu.sync_copy(e_hbm.at[pl.ds(j0, pairs_w)], e_v)
        pltpu.sync_copy(ps_hbm, ps_v)
        lanes = lax.iota(jnp.int32, chunk)
        bufs = (buf0, buf1)
        gsems = (gs0, gs1)
        ssems = (ss0, ss1)

        def gstart(c, b):
            return pltpu.async_copy(
                x_hbm.at[tok_v.at[pl.ds(c * chunk, chunk)]],
                bufs[b], gsems[b])

        def sstart(c, b):
            ev = e_v[pl.ds(c * chunk, chunk)]
            dst = plsc.load_gather(ps_v, [ev]) + lanes + (j0 + c * chunk)
            return pltpu.async_copy(bufs[b], out_hbm.at[dst], ssems[b])

        gd = [None] * n_chunks
        sd = [None] * n_chunks
        gd[0] = gstart(0, 0)
        for c in range(n_chunks):
            b = c & 1
            gd[c].wait()
            if c >= 1:
                sd[c - 1].wait()
            if c + 1 < n_chunks:
                gd[c + 1] = gstart(c + 1, 1 - b)
            sd[c] = sstart(c, b)
        sd[n_chunks - 1].wait()

    return k(x, tok_sorted, sorted_e, padshift)


def _ffn_body(eot_ref, act_ref, xsblk_ref, xs_ref, w_ref, w1_ref, w3_ref,
              w2_ref, out_ref):
    i = pl.program_id(0)

    @pl.when(act_ref[i] == 1)
    def _():
        x = xs_ref[...].astype(jnp.float32)
        g = jnp.dot(x, w1_ref[0], preferred_element_type=jnp.float32)
        u = jnp.dot(x, w3_ref[0], preferred_element_type=jnp.float32)
        h = (g * jax.nn.sigmoid(g)) * u
        y = jnp.dot(h, w2_ref[0], preferred_element_type=jnp.float32)
        out_ref[...] = y * w_ref[...]

    @pl.when(act_ref[i] == 0)
    def _():
        out_ref[...] = jnp.zeros_like(out_ref)


def _ffn(xs, w_padded, W1, W3, W2, eot, active, xsblk, P, NT):
    H = xs.shape[1]
    FF = W1.shape[2]
    grid_spec = pltpu.PrefetchScalarGridSpec(
        num_scalar_prefetch=3,
        grid=(NT,),
        in_specs=[
            pl.BlockSpec((_TM, H), lambda i, eot, act, xb: (xb[i], 0)),
            pl.BlockSpec((_TM, 1), lambda i, eot, act, xb: (xb[i], 0)),
            pl.BlockSpec((1, H, FF), lambda i, eot, act, xb: (eot[i], 0, 0)),
            pl.BlockSpec((1, H, FF), lambda i, eot, act, xb: (eot[i], 0, 0)),
            pl.BlockSpec((1, FF, H), lambda i, eot, act, xb: (eot[i], 0, 0)),
        ],
        out_specs=pl.BlockSpec((_TM, H), lambda i, eot, act, xb: (i, 0)),
    )
    return pl.pallas_call(
        _ffn_body,
        grid_spec=grid_spec,
        out_shape=jax.ShapeDtypeStruct((P, H), jnp.float32),
        compiler_params=pltpu.CompilerParams(
            dimension_semantics=("arbitrary",),
        ),
    )(eot, active, xsblk, xs, w_padded.reshape(P, 1), W1, W3, W2)


def _combine(ys, pos, T, K, H):
    toks_w = T // _NW
    BT = 2
    rounds = toks_w // BT
    mesh = plsc.VectorSubcoreMesh(core_axis_name="c", subcore_axis_name="s",
                                  num_cores=_NC, num_subcores=_NS)

    @functools.partial(
        pl.kernel, mesh=mesh,
        out_type=jax.ShapeDtypeStruct((T, H), jnp.float32),
        scratch_types=[
            pltpu.VMEM((toks_w * K,), jnp.int32),
            pltpu.VMEM((BT * K, H), jnp.float32),
            pltpu.VMEM((BT * K, H), jnp.float32),
            pltpu.VMEM((BT, H), jnp.float32),
            pltpu.VMEM((BT, H), jnp.float32),
            pltpu.SemaphoreType.DMA,
            pltpu.SemaphoreType.DMA,
            pltpu.SemaphoreType.DMA,
            pltpu.SemaphoreType.DMA,
        ],
    )
    def k(ys_hbm, pos_hbm, out_hbm, idx_v, b0, b1, o0, o1,
          gs0, gs1, ss0, ss1):
        wid = lax.axis_index("s") * _NC + lax.axis_index("c")
        base = wid * toks_w
        pltpu.sync_copy(pos_hbm.at[pl.ds(base * K, toks_w * K)], idx_v)

        def gsrc(r):
            return ys_hbm.at[idx_v.at[pl.ds(r * BT * K, BT * K)]]

        def odst(r):
            return out_hbm.at[pl.ds(base + r * BT, BT)]

        def accum(buf, ost):
            def col_body(i, _):
                for tt in range(BT):
                    s = buf[tt * K, pl.ds(i * 16, 16)]
                    for kk in range(1, K):
                        s = s + buf[tt * K + kk, pl.ds(i * 16, 16)]
                    ost[tt, pl.ds(i * 16, 16)] = s
                return _
            lax.fori_loop(0, H // 16, col_body, None)

        pltpu.async_copy(gsrc(0), b0, gs0)
        pltpu.async_copy(gsrc(1), b1, gs1)

        def body(i, _):
            for par, (buf, ost, gs, ss) in enumerate(
                    ((b0, o0, gs0, ss0), (b1, o1, gs1, ss1))):
                r = 2 * i + par
                pltpu.make_async_copy(gsrc(r), buf, gs).wait()

                @pl.when(i > 0)
                def _():
                    pltpu.make_async_copy(ost, odst(r), ss).wait()

                accum(buf, ost)
                pltpu.async_copy(ost, odst(r), ss)

                @pl.when(r + 2 < rounds)
                def _():
                    pltpu.async_copy(gsrc(r + 2), buf, gs)
            return _

        lax.fori_loop(0, rounds // 2, body, None)
        pltpu.make_async_copy(o0, odst(rounds - 2), ss0).wait()
        pltpu.make_async_copy(o1, odst(rounds - 1), ss1).wait()

    return k(ys, pos)


def kernel(hidden_states, Wg, W1, W3, W2):
    orig_shape = hidden_states.shape
    H = orig_shape[-1]
    E = Wg.shape[1]
    K = 8
    x = hidden_states.reshape(-1, H)
    T = x.shape[0]

    P = ((T * K + E * (_TM - 1) + _TM - 1) // _TM) * _TM
    NT = P // _TM

    topk_w, topk_i = _router(x, Wg, K)
    (tok_sorted, sorted_e, padshift, w_padded, pos, eot, active,
     xsblk) = _routing_metadata(topk_w, topk_i, E, P, NT)
    xs = _dispatch_gather(x, tok_sorted, sorted_e, padshift, P, E)
    ys = _ffn(xs, w_padded, W1, W3, W2, eot, active, xsblk, P, NT)
    out = _combine(ys, pos, T, K, H)
    return out.reshape(orig_shape)
